# baseline (speedup 1.0000x reference)
import functools

import jax
import jax.numpy as jnp
from jax import lax
from jax.experimental import pallas as pl
from jax.experimental.pallas import tpu as pltpu
from jax.experimental.pallas import tpu_sc as plsc

PLACEHOLDER = -1
VOCAB = 100000
SPEC = 4
BATCH = 32
L = 16
CHUNK = 10000
NCHUNK = VOCAB // CHUNK
NVREG = CHUNK // L
OUTW = 8
INT_MAX = 2**31 - 1


def _sc_body(t_hbm, d_hbm, q_hbm, ids_hbm, uni_hbm, bon_hbm, gre_hbm,
             out_hbm,
             tb0, tb1, db0, db1, qb0, qb1,
             ids_v, uni_v, bon_v, gre_v, idx_v, ga_v, gb_v, out_v,
             sem0, sem1, gsem):
    b = lax.axis_index("c") * 16 + lax.axis_index("s")
    iota = lax.iota(jnp.int32, L)

    pltpu.sync_copy(ids_hbm, ids_v)
    pltpu.sync_copy(uni_hbm, uni_v)
    pltpu.sync_copy(bon_hbm, bon_v)
    pltpu.sync_copy(gre_hbm, gre_v)

    tbufs = (tb0, tb1)
    dbufs = (db0, db1)
    qbufs = (qb0, qb1)
    sems = (sem0, sem1)

    def issue(base, ci):
        off = base + ci * CHUNK
        k = ci % 2
        return (
            pltpu.async_copy(t_hbm.at[pl.ds(off, CHUNK)], tbufs[k], sems[k]),
            pltpu.async_copy(d_hbm.at[pl.ds(off, CHUNK)], dbufs[k], sems[k]),
            pltpu.async_copy(q_hbm.at[pl.ds(off, CHUNK)], qbufs[k], sems[k]),
        )

    rec_vec = jnp.zeros((L,), jnp.int32)

    for r in range(SPEC):
        base = (b * SPEC + r) * VOCAB
        pending = {0: issue(base, 0)}
        bd = jnp.full((L,), -1.0, jnp.float32)
        bq = jnp.ones((L,), jnp.float32)
        bi = jnp.zeros((L,), jnp.int32)
        for ci in range(NCHUNK):
            if ci + 1 < NCHUNK:
                pending[ci + 1] = issue(base, ci + 1)
            for h in pending.pop(ci):
                h.wait()
            tb, db, qb = tbufs[ci % 2], dbufs[ci % 2], qbufs[ci % 2]

            def body(j, carry, tb=tb, db=db, qb=qb):
                bd, bq, bi, idx = carry
                t = tb[pl.ds(j * L, L)]
                d = db[pl.ds(j * L, L)]
                qv = qb[pl.ds(j * L, L)]
                diff = jnp.maximum(t - d, 0.0)
                better = diff * bq > bd * qv
                bd = jnp.where(better, diff, bd)
                bq = jnp.where(better, qv, bq)
                bi = jnp.where(better, idx, bi)
                return bd, bq, bi, idx + L

            bd, bq, bi, _ = lax.fori_loop(
                0, NVREG, body, (bd, bq, bi, ci * CHUNK + iota), unroll=4)
        val = bd / bq
        idx = bi
        for s in (1, 2, 4, 8):
            ga_v[...] = val
            idx_v[...] = idx
            pv = plsc.load_gather(ga_v, [iota ^ s])
            pi = plsc.load_gather(idx_v, [iota ^ s])
            better = (pv > val) | ((pv == val) & (pi < idx))
            val = jnp.where(better, pv, val)
            idx = jnp.where(better, pi, idx)
        rec_vec = jnp.where(iota == r, idx, rec_vec)

    lane_r = jnp.minimum(iota, SPEC - 1)
    row_idx = b * SPEC + lane_r
    dtok = plsc.load_gather(ids_v, [row_idx])
    u = plsc.load_gather(uni_v, [row_idx])
    idx_v[...] = row_idx * VOCAB + dtok
    pltpu.async_copy(d_hbm.at[idx_v], ga_v, gsem).wait()
    pltpu.async_copy(t_hbm.at[idx_v], gb_v, gsem).wait()
    dp = ga_v[...]
    tp = gb_v[...]

    accept = (dp > 0.0) & ((tp / jnp.maximum(dp, 1e-30)) >= u)
    accept = jnp.where(iota >= SPEC, True, accept)
    bvec = plsc.load_gather(bon_v, [jnp.full((L,), b, jnp.int32)])
    gvec = plsc.load_gather(gre_v, [jnp.full((L,), b, jnp.int32)])
    dtok = jnp.where(iota == SPEC, bvec, dtok)

    rej = jnp.where(accept, 0, 1)
    cum = rej
    for s in (1, 2, 4):
        idx_v[...] = cum
        sh = plsc.load_gather(idx_v, [jnp.maximum(iota - s, 0)])
        cum = cum + jnp.where(iota >= s, sh, 0)
    excl = cum - rej
    tok = jnp.where(accept, dtok, rec_vec)
    outv = jnp.where(excl == 0, tok, PLACEHOLDER)
    outv = jnp.where(gvec > 0, PLACEHOLDER, outv)
    out_v[...] = outv
    pltpu.sync_copy(out_v.at[pl.ds(0, OUTW)], out_hbm.at[pl.ds(b * OUTW, OUTW)])


@functools.lru_cache(maxsize=1)
def _build():
    mesh = plsc.VectorSubcoreMesh(core_axis_name="c", subcore_axis_name="s")
    return pl.kernel(
        _sc_body,
        out_type=jax.ShapeDtypeStruct((BATCH * OUTW,), jnp.int32),
        mesh=mesh,
        compiler_params=pltpu.CompilerParams(needs_layout_passes=False),
        scratch_types=[
            pltpu.VMEM((CHUNK,), jnp.float32),
            pltpu.VMEM((CHUNK,), jnp.float32),
            pltpu.VMEM((CHUNK,), jnp.float32),
            pltpu.VMEM((CHUNK,), jnp.float32),
            pltpu.VMEM((CHUNK,), jnp.float32),
            pltpu.VMEM((CHUNK,), jnp.float32),
            pltpu.VMEM((BATCH * SPEC,), jnp.int32),
            pltpu.VMEM((BATCH * SPEC,), jnp.float32),
            pltpu.VMEM((BATCH,), jnp.int32),
            pltpu.VMEM((BATCH,), jnp.int32),
            pltpu.VMEM((L,), jnp.int32),
            pltpu.VMEM((L,), jnp.float32),
            pltpu.VMEM((L,), jnp.float32),
            pltpu.VMEM((L,), jnp.int32),
            pltpu.SemaphoreType.DMA,
            pltpu.SemaphoreType.DMA,
            pltpu.SemaphoreType.DMA,
        ],
    )


def kernel(draft_token_ids, draft_probs, target_probs, bonus_token_ids,
           uniform_probs, q, cu_num_draft_tokens, is_greedy):
    del cu_num_draft_tokens
    tflat = target_probs.reshape(-1)
    dflat = draft_probs.reshape(-1)
    qflat = q.reshape(-1)
    ids = draft_token_ids.astype(jnp.int32)
    bon = bonus_token_ids.reshape(-1).astype(jnp.int32)
    gre = is_greedy.astype(jnp.int32)
    out = _build()(tflat, dflat, qflat, ids, uniform_probs, bon, gre)
    out = out.reshape(BATCH, OUTW)[:, :SPEC + 1]
    return out.astype(draft_token_ids.dtype)


# speedup vs baseline: 1.0057x; 1.0057x over previous
import functools

import jax
import jax.numpy as jnp
from jax import lax
from jax.experimental import pallas as pl
from jax.experimental.pallas import tpu as pltpu
from jax.experimental.pallas import tpu_sc as plsc

PLACEHOLDER = -1
VOCAB = 100000
SPEC = 4
BATCH = 32
L = 16
CHUNK = 10000
NCHUNK = VOCAB // CHUNK
NVREG = CHUNK // L
OUTW = 8


def _sc_body(t_hbm, d_hbm, q_hbm, ids_hbm, uni_hbm, bon_hbm, gre_hbm,
             out_hbm,
             tb0, tb1, db0, db1, qb0, qb1,
             ids_v, uni_v, bon_v, gre_v, tmpi_v, tmpf_v, out_v, rowidx_v,
             sem0, sem1):
    b = lax.axis_index("c") * 16 + lax.axis_index("s")
    iota = lax.iota(jnp.int32, L)

    pltpu.sync_copy(ids_hbm, ids_v)
    pltpu.sync_copy(uni_hbm, uni_v)
    pltpu.sync_copy(bon_hbm, bon_v)
    pltpu.sync_copy(gre_hbm, gre_v)

    zero16 = jnp.zeros((L,), jnp.int32)
    base = b * SPEC * NCHUNK
    nstep = SPEC * NCHUNK
    for k in range((nstep + L - 1) // L):
        pos = k * L + iota
        plsc.store_scatter(rowidx_v, [jnp.minimum(pos, nstep - 1), zero16],
                           base + pos, mask=pos < nstep)

    tbufs = (tb0, tb1)
    dbufs = (db0, db1)
    qbufs = (qb0, qb1)
    sems = (sem0, sem1)

    def issue(step, k):
        idx = rowidx_v.at[step]
        return (
            pltpu.async_copy(t_hbm.at[idx], tbufs[k], sems[k]),
            pltpu.async_copy(d_hbm.at[idx], dbufs[k], sems[k]),
            pltpu.async_copy(q_hbm.at[idx], qbufs[k], sems[k]),
        )

    rec_vec = jnp.zeros((L,), jnp.int32)
    dpacc = jnp.zeros((L,), jnp.float32)
    tpacc = jnp.zeros((L,), jnp.float32)

    for r in range(SPEC):
        id_vec = plsc.load_gather(ids_v, [jnp.full((L,), b * SPEC + r,
                                                   jnp.int32)])
        pending = {0: issue(r * NCHUNK, 0)}
        bd = jnp.full((L,), -1.0, jnp.float32)
        bq = jnp.ones((L,), jnp.float32)
        bi = jnp.zeros((L,), jnp.int32)
        for ci in range(NCHUNK):
            if ci + 1 < NCHUNK:
                pending[ci + 1] = issue(r * NCHUNK + ci + 1, (ci + 1) % 2)
            for h in pending.pop(ci):
                h.wait()
            tb, db, qb = tbufs[ci % 2], dbufs[ci % 2], qbufs[ci % 2]

            off = id_vec - ci * CHUNK
            inchunk = (off >= 0) & (off < CHUNK)
            offc = jnp.clip(off, 0, CHUNK - 1)
            gd = plsc.load_gather(db, [zero16, offc])
            gt = plsc.load_gather(tb, [zero16, offc])
            sel = inchunk & (iota == r)
            dpacc = jnp.where(sel, gd, dpacc)
            tpacc = jnp.where(sel, gt, tpacc)

            def body(j, carry, tb=tb, db=db, qb=qb):
                bd, bq, bi, idx = carry
                t = tb[0, pl.ds(j * L, L)]
                d = db[0, pl.ds(j * L, L)]
                qv = qb[0, pl.ds(j * L, L)]
                diff = jnp.maximum(t - d, 0.0)
                better = diff * bq > bd * qv
                bd = jnp.where(better, diff, bd)
                bq = jnp.where(better, qv, bq)
                bi = jnp.where(better, idx, bi)
                return bd, bq, bi, idx + L

            bd, bq, bi, _ = lax.fori_loop(
                0, NVREG, body, (bd, bq, bi, ci * CHUNK + iota), unroll=4)
        val = bd / bq
        idx = bi
        for s in (1, 2, 4, 8):
            tmpf_v[...] = val
            tmpi_v[...] = idx
            pv = plsc.load_gather(tmpf_v, [iota ^ s])
            pi = plsc.load_gather(tmpi_v, [iota ^ s])
            better = (pv > val) | ((pv == val) & (pi < idx))
            val = jnp.where(better, pv, val)
            idx = jnp.where(better, pi, idx)
        rec_vec = jnp.where(iota == r, idx, rec_vec)

    lane_r = jnp.minimum(iota, SPEC - 1)
    row_idx = b * SPEC + lane_r
    dtok = plsc.load_gather(ids_v, [row_idx])
    u = plsc.load_gather(uni_v, [row_idx])
    accept = (dpacc > 0.0) & ((tpacc / jnp.maximum(dpacc, 1e-30)) >= u)
    accept = jnp.where(iota >= SPEC, True, accept)
    bvec = plsc.load_gather(bon_v, [jnp.full((L,), b, jnp.int32)])
    gvec = plsc.load_gather(gre_v, [jnp.full((L,), b, jnp.int32)])
    dtok = jnp.where(iota == SPEC, bvec, dtok)

    rej = jnp.where(accept, 0, 1)
    cum = rej
    for s in (1, 2, 4):
        tmpi_v[...] = cum
        sh = plsc.load_gather(tmpi_v, [jnp.maximum(iota - s, 0)])
        cum = cum + jnp.where(iota >= s, sh, 0)
    excl = cum - rej
    tok = jnp.where(accept, dtok, rec_vec)
    outv = jnp.where(excl == 0, tok, PLACEHOLDER)
    outv = jnp.where(gvec > 0, PLACEHOLDER, outv)
    out_v[...] = outv
    pltpu.sync_copy(out_v.at[pl.ds(0, OUTW)], out_hbm.at[pl.ds(b * OUTW, OUTW)])


@functools.lru_cache(maxsize=1)
def _build():
    mesh = plsc.VectorSubcoreMesh(core_axis_name="c", subcore_axis_name="s")
    return pl.kernel(
        _sc_body,
        out_type=jax.ShapeDtypeStruct((BATCH * OUTW,), jnp.int32),
        mesh=mesh,
        compiler_params=pltpu.CompilerParams(needs_layout_passes=False,
                                             use_tc_tiling_on_sc=False),
        scratch_types=[
            pltpu.VMEM((1, CHUNK), jnp.float32),
            pltpu.VMEM((1, CHUNK), jnp.float32),
            pltpu.VMEM((1, CHUNK), jnp.float32),
            pltpu.VMEM((1, CHUNK), jnp.float32),
            pltpu.VMEM((1, CHUNK), jnp.float32),
            pltpu.VMEM((1, CHUNK), jnp.float32),
            pltpu.VMEM((BATCH * SPEC,), jnp.int32),
            pltpu.VMEM((BATCH * SPEC,), jnp.float32),
            pltpu.VMEM((BATCH,), jnp.int32),
            pltpu.VMEM((BATCH,), jnp.int32),
            pltpu.VMEM((L,), jnp.int32),
            pltpu.VMEM((L,), jnp.float32),
            pltpu.VMEM((L,), jnp.int32),
            pltpu.VMEM((3 * L, 1), jnp.int32),
            pltpu.SemaphoreType.DMA,
            pltpu.SemaphoreType.DMA,
        ],
    )


def kernel(draft_token_ids, draft_probs, target_probs, bonus_token_ids,
           uniform_probs, q, cu_num_draft_tokens, is_greedy):
    del cu_num_draft_tokens
    t2 = target_probs.reshape(-1, CHUNK)
    d2 = draft_probs.reshape(-1, CHUNK)
    q2 = q.reshape(-1, CHUNK)
    ids = draft_token_ids.astype(jnp.int32)
    bon = bonus_token_ids.reshape(-1).astype(jnp.int32)
    gre = is_greedy.astype(jnp.int32)
    out = _build()(t2, d2, q2, ids, uniform_probs, bon, gre)
    out = out.reshape(BATCH, OUTW)[:, :SPEC + 1]
    return out.astype(draft_token_ids.dtype)


# speedup vs baseline: 1.6144x; 1.6053x over previous
import functools

import jax
import jax.numpy as jnp
from jax import lax
from jax.experimental import pallas as pl
from jax.experimental.pallas import tpu as pltpu
from jax.experimental.pallas import tpu_sc as plsc

PLACEHOLDER = -1
VOCAB = 100000
SPEC = 4
BATCH = 32
NROW = BATCH * SPEC
L = 16

V_TC = 61056
TC_CB = 6784
TC_NG = V_TC // TC_CB
TAIL = 99968
TAILW = VOCAB - TAIL
SC_HALF_T = (TAIL - V_TC) // 128 // 2
NT = 8
SC_NCH = SC_HALF_T // NT
INT_MAX = 2**31 - 1


def _sc_body(t_hbm, d_hbm, q_hbm, ids_hbm,
             oval, oidx, opp,
             tb0, tb1, db0, db1, qb0, qb1,
             ids_v, tmpi_v, tmpf_v,
             sem0, sem1):
    c = lax.axis_index("c")
    s = lax.axis_index("s")
    gp = c * 8 + s // 2
    h = s % 2
    row0 = pl.multiple_of(gp * 8, 8)
    shard_col = V_TC + h * (SC_HALF_T * 128)
    iota = lax.iota(jnp.int32, L)
    lane8 = iota & 7

    pltpu.sync_copy(ids_hbm, ids_v)
    idv = plsc.load_gather(ids_v, [row0 + lane8])

    tbufs = (tb0, tb1)
    dbufs = (db0, db1)
    qbufs = (qb0, qb1)
    sems = (sem0, sem1)

    def chunk_col(ci):
        return shard_col + jnp.minimum(ci, SC_NCH - 1) * (NT * 128)

    def copies(ci, k):
        col = chunk_col(ci)
        cps = []
        for kt in range(NT):
            c2 = pl.multiple_of(col + kt * 128, 128)
            src = (pl.ds(row0, 8), pl.ds(c2, 128))
            cps.append(pltpu.make_async_copy(
                t_hbm.at[src], tbufs[k].at[kt], sems[k]))
            cps.append(pltpu.make_async_copy(
                d_hbm.at[src], dbufs[k].at[kt], sems[k]))
            cps.append(pltpu.make_async_copy(
                q_hbm.at[src], qbufs[k].at[kt], sems[k]))
        return cps

    def issue(ci, k):
        for cp in copies(ci, k):
            cp.start()

    def drain(ci, k):
        for cp in copies(ci, k):
            cp.wait()

    accs = []
    for _ in range(8):
        accs += [jnp.full((L,), -1.0, jnp.float32),
                 jnp.ones((L,), jnp.float32),
                 jnp.zeros((L,), jnp.int32)]
    ppacc = jnp.zeros((L,), jnp.float32)

    issue(0, 0)
    issue(1, 1)

    def do_chunk(ci, k, carry):
        accs = list(carry[:-1])
        ppacc = carry[-1]
        drain(ci, k)
        tb, db, qb = tbufs[k], dbufs[k], qbufs[k]
        colbase = chunk_col(ci)

        off = idv - colbase
        inchunk = (off >= 0) & (off < NT * 128)
        offc = jnp.clip(off, 0, NT * 128 - 1)
        kti = offc >> 7
        li = offc & 127
        gd = plsc.load_gather(db, [kti, lane8, li])
        gt = plsc.load_gather(tb, [kti, lane8, li])
        g = jnp.where(iota < 8, gd, gt)
        ppacc = jnp.where(inchunk, g, ppacc)

        def tbody(kt, pc):
            pc = list(pc)
            idxk = pc[-1]
            for p in range(8):
                idx = idxk + p * L
                for r in range(8):
                    t = tb[kt, r, pl.ds(p * L, L)]
                    d = db[kt, r, pl.ds(p * L, L)]
                    qv = qb[kt, r, pl.ds(p * L, L)]
                    diff = jnp.maximum(t - d, 0.0)
                    bd, bq, bi = pc[3 * r], pc[3 * r + 1], pc[3 * r + 2]
                    better = diff * bq > bd * qv
                    pc[3 * r] = jnp.where(better, diff, bd)
                    pc[3 * r + 1] = jnp.where(better, qv, bq)
                    pc[3 * r + 2] = jnp.where(better, idx, bi)
            pc[-1] = idxk + 128
            return tuple(pc)

        res = lax.fori_loop(0, NT, tbody, tuple(accs) + (colbase + iota,))
        return list(res[:-1]) + [ppacc]

    def two_chunks(i, carry):
        carry = do_chunk(2 * i, 0, carry)

        @pl.when(2 * i + 2 < SC_NCH + 1)
        def _():
            issue(2 * i + 2, 0)

        carry = do_chunk(2 * i + 1, 1, carry)

        @pl.when(2 * i + 3 < SC_NCH + 1)
        def _():
            issue(2 * i + 3, 1)
        return tuple(carry)

    res = lax.fori_loop(0, (SC_NCH + 1) // 2, two_chunks,
                        tuple(accs) + (ppacc,))
    accs = list(res[:-1])
    ppacc = res[-1]

    valv = jnp.zeros((L,), jnp.float32)
    biv = jnp.zeros((L,), jnp.int32)
    for r in range(8):
        bd, bq, bi = accs[3 * r], accs[3 * r + 1], accs[3 * r + 2]
        val = bd / bq
        idx = bi
        for st in (1, 2, 4, 8):
            tmpf_v[...] = val
            tmpi_v[...] = idx
            pv = plsc.load_gather(tmpf_v, [iota ^ st])
            pi = plsc.load_gather(tmpi_v, [iota ^ st])
            better = (pv > val) | ((pv == val) & (pi < idx))
            val = jnp.where(better, pv, val)
            idx = jnp.where(better, pi, idx)
        valv = jnp.where(iota == r, val, valv)
        biv = jnp.where(iota == r, idx, biv)

    woff = (h * 16 + gp) * L
    tmpf_v[...] = valv
    pltpu.sync_copy(tmpf_v, oval.at[pl.ds(woff, L)])
    tmpi_v[...] = biv
    pltpu.sync_copy(tmpi_v, oidx.at[pl.ds(woff, L)])
    tmpf_v[...] = ppacc
    pltpu.sync_copy(tmpf_v, opp.at[pl.ds(woff, L)])


@functools.lru_cache(maxsize=1)
def _build_sc():
    mesh = plsc.VectorSubcoreMesh(core_axis_name="c", subcore_axis_name="s")
    return pl.kernel(
        _sc_body,
        out_type=(
            jax.ShapeDtypeStruct((512,), jnp.float32),
            jax.ShapeDtypeStruct((512,), jnp.int32),
            jax.ShapeDtypeStruct((512,), jnp.float32),
        ),
        mesh=mesh,
        compiler_params=pltpu.CompilerParams(needs_layout_passes=False),
        scratch_types=[
            pltpu.VMEM((NT, 8, 128), jnp.float32),
            pltpu.VMEM((NT, 8, 128), jnp.float32),
            pltpu.VMEM((NT, 8, 128), jnp.float32),
            pltpu.VMEM((NT, 8, 128), jnp.float32),
            pltpu.VMEM((NT, 8, 128), jnp.float32),
            pltpu.VMEM((NT, 8, 128), jnp.float32),
            pltpu.VMEM((NROW,), jnp.int32),
            pltpu.VMEM((L,), jnp.int32),
            pltpu.VMEM((L,), jnp.float32),
            pltpu.SemaphoreType.DMA,
            pltpu.SemaphoreType.DMA,
        ],
    )


def _tc_scan_body(ids_ref, t_ref, d_ref, q_ref, m_ref, i_ref, dp_ref, tp_ref):
    g = pl.program_id(0)
    base = g * TC_CB
    t = t_ref[...]
    d = d_ref[...]
    qv = q_ref[...]
    val = jnp.maximum(t - d, 0.0) / qv
    cm = jnp.max(val, axis=1, keepdims=True)
    col = lax.broadcasted_iota(jnp.int32, (NROW, TC_CB), 1)
    ca = jnp.min(jnp.where(val == cm, col, INT_MAX), axis=1,
                 keepdims=True) + base

    hit = col == ids_ref[...] - base
    dpc = jnp.sum(jnp.where(hit, d, 0.0), axis=1, keepdims=True)
    tpc = jnp.sum(jnp.where(hit, t, 0.0), axis=1, keepdims=True)

    first = g == 0
    m_prev = jnp.where(first, -1.0, m_ref[...])
    i_prev = jnp.where(first, 0, i_ref[...])
    dp_prev = jnp.where(first, 0.0, dp_ref[...])
    tp_prev = jnp.where(first, 0.0, tp_ref[...])
    better = cm > m_prev
    m_ref[...] = jnp.where(better, cm, m_prev)
    i_ref[...] = jnp.where(better, ca, i_prev)
    dp_ref[...] = dp_prev + dpc
    tp_ref[...] = tp_prev + tpc


def _tc_scan(ids, t, d, q):
    vec = pl.BlockSpec((NROW, 1), lambda i: (0, 0))
    blk = pl.BlockSpec((NROW, TC_CB), lambda i: (0, i))
    return pl.pallas_call(
        _tc_scan_body,
        grid=(TC_NG,),
        in_specs=[vec, blk, blk, blk],
        out_specs=[vec, vec, vec, vec],
        out_shape=[
            jax.ShapeDtypeStruct((NROW, 1), jnp.float32),
            jax.ShapeDtypeStruct((NROW, 1), jnp.int32),
            jax.ShapeDtypeStruct((NROW, 1), jnp.float32),
            jax.ShapeDtypeStruct((NROW, 1), jnp.float32),
        ],
        compiler_params=pltpu.CompilerParams(
            dimension_semantics=("arbitrary",)),
    )(ids, t, d, q)


def _tail_body(ids_ref, t_ref, d_ref, q_ref, m_ref, i_ref, dp_ref, tp_ref):
    col = lax.broadcasted_iota(jnp.int32, (NROW, 128), 1)
    tvalid = col < TAILW
    t = t_ref[...]
    d = d_ref[...]
    val = jnp.maximum(t - d, 0.0) / q_ref[...]
    val = jnp.where(tvalid, val, -1.0)
    cm = jnp.max(val, axis=1, keepdims=True)
    m_ref[...] = cm
    i_ref[...] = jnp.min(jnp.where(val == cm, col, INT_MAX), axis=1,
                         keepdims=True) + TAIL
    hit = (col == ids_ref[...] - TAIL) & tvalid
    dp_ref[...] = jnp.sum(jnp.where(hit, d, 0.0), axis=1, keepdims=True)
    tp_ref[...] = jnp.sum(jnp.where(hit, t, 0.0), axis=1, keepdims=True)


def _tc_tail(ids, t, d, q):
    vec = pl.BlockSpec((NROW, 1), lambda i: (0, 0))
    tailspec = pl.BlockSpec((NROW, 128), lambda i: (0, TAIL // 128))
    return pl.pallas_call(
        _tail_body,
        grid=(1,),
        in_specs=[vec, tailspec, tailspec, tailspec],
        out_specs=[vec, vec, vec, vec],
        out_shape=[
            jax.ShapeDtypeStruct((NROW, 1), jnp.float32),
            jax.ShapeDtypeStruct((NROW, 1), jnp.int32),
            jax.ShapeDtypeStruct((NROW, 1), jnp.float32),
            jax.ShapeDtypeStruct((NROW, 1), jnp.float32),
        ],
    )(ids, t, d, q)


def _merge_body(tcm, tci, tcdp, tctp, sv0, si0, sdp0, stp0,
                sv1, si1, sdp1, stp1, tm, ta, tdp, ttp,
                ids, uni, bon, gre, out_ref):
    m, i = tcm[...], tci[...]
    for hv, hi in ((sv0[...], si0[...]), (sv1[...], si1[...]),
                   (tm[...], ta[...])):
        win = (hv > m) | ((hv == m) & (hi < i))
        m = jnp.where(win, hv, m)
        i = jnp.where(win, hi, i)
    dp = tcdp[...] + sdp0[...] + sdp1[...] + tdp[...]
    tp = tctp[...] + stp0[...] + stp1[...] + ttp[...]

    accm = (dp > 0.0) & ((tp / jnp.maximum(dp, 1e-30)) >= uni[...])
    token = jnp.where(accm, ids[...], i)
    cols = []
    cum = jnp.ones((BATCH, 1), jnp.bool_)
    for k in range(SPEC):
        cols.append(jnp.where(cum, token[:, k:k + 1], PLACEHOLDER))
        cum = cum & accm[:, k:k + 1]
    cols.append(jnp.where(cum, bon[...], PLACEHOLDER))
    out = jnp.concatenate(cols, axis=1)
    out_ref[...] = jnp.where(gre[...] > 0, PLACEHOLDER, out)


def _merge(*args):
    m4 = pl.BlockSpec((BATCH, SPEC), lambda i: (0, 0))
    m1 = pl.BlockSpec((BATCH, 1), lambda i: (0, 0))
    return pl.pallas_call(
        _merge_body,
        grid=(1,),
        in_specs=[m4] * 18 + [m1, m1],
        out_specs=pl.BlockSpec((BATCH, SPEC + 1), lambda i: (0, 0)),
        out_shape=jax.ShapeDtypeStruct((BATCH, SPEC + 1), jnp.int32),
    )(*args)


def kernel(draft_token_ids, draft_probs, target_probs, bonus_token_ids,
           uniform_probs, q, cu_num_draft_tokens, is_greedy):
    del cu_num_draft_tokens
    ids = draft_token_ids.astype(jnp.int32)
    bon = bonus_token_ids.reshape(-1).astype(jnp.int32)
    gre = is_greedy.astype(jnp.int32)

    ids2 = ids.reshape(NROW, 1)
    oval, oidx, opp = _build_sc()(target_probs, draft_probs, q, ids)
    r4 = lambda x: x.reshape(16, L)[:, :8].reshape(BATCH, SPEC)
    sv = oval.reshape(2, 16 * L)
    si = oidx.reshape(2, 16 * L)
    pp = opp.reshape(2, 16, L)
    sv0, sv1 = r4(sv[0]), r4(sv[1])
    si0, si1 = r4(si[0]), r4(si[1])
    sdp0 = pp[0, :, :8].reshape(BATCH, SPEC)
    sdp1 = pp[1, :, :8].reshape(BATCH, SPEC)
    stp0 = pp[0, :, 8:].reshape(BATCH, SPEC)
    stp1 = pp[1, :, 8:].reshape(BATCH, SPEC)

    tcm, tci, tcdp, tctp = _tc_scan(ids2, target_probs, draft_probs, q)
    tm, ta, tdp, ttp = _tc_tail(ids2, target_probs, draft_probs, q)

    r = lambda x: x.reshape(BATCH, SPEC)
    out = _merge(r(tcm), r(tci), r(tcdp), r(tctp),
                 sv0, si0, sdp0, stp0, sv1, si1, sdp1, stp1,
                 r(tm), r(ta), r(tdp), r(ttp),
                 r(ids), r(uniform_probs),
                 bon.reshape(BATCH, 1), gre.reshape(BATCH, 1))
    return out.astype(draft_token_ids.dtype)


# speedup vs baseline: 3.4453x; 2.1340x over previous
import functools

import jax
import jax.numpy as jnp
from jax import lax
from jax.experimental import pallas as pl
from jax.experimental.pallas import tpu as pltpu
from jax.experimental.pallas import tpu_sc as plsc

PLACEHOLDER = -1
VOCAB = 100000
SPEC = 4
BATCH = 32
NROW = BATCH * SPEC
L = 16
INT_MAX = 2**31 - 1

V_TC = 59040
TC_B0 = 1968
TC_NG = V_TC // TC_B0
SC_ROWS = VOCAB - V_TC
W_ROWS = SC_ROWS // 32
NT = 8
CROWS = NT * 8
SC_NCH = W_ROWS // CROWS


def _sc_body(t_hbm, d_hbm, q_hbm, ids_hbm,
             oval, oidx, opp,
             tb0, tb1, db0, db1, qb0, qb1,
             ids_v, tmpf_v, obuf_v, obi_v, idxg_v, gt_v, gd_v,
             sem0, sem1, gsem):
    c = lax.axis_index("c")
    s = lax.axis_index("s")
    w = c * 16 + s
    rs = V_TC + w * W_ROWS
    iota = lax.iota(jnp.int32, L)

    pltpu.sync_copy(ids_hbm, ids_v)

    tbufs = (tb0, tb1)
    dbufs = (db0, db1)
    qbufs = (qb0, qb1)
    sems = (sem0, sem1)

    def copies(ci, k):
        base = rs + ci * CROWS
        cps = []
        for kt in range(NT):
            r0 = pl.multiple_of(base + kt * 8, 8)
            cps.append(pltpu.make_async_copy(
                t_hbm.at[pl.ds(r0, 8)], tbufs[k].at[kt], sems[k]))
            cps.append(pltpu.make_async_copy(
                d_hbm.at[pl.ds(r0, 8)], dbufs[k].at[kt], sems[k]))
            cps.append(pltpu.make_async_copy(
                q_hbm.at[pl.ds(r0, 8)], qbufs[k].at[kt], sems[k]))
        return cps

    def issue(ci, k):
        for cp in copies(ci, k):
            cp.start()

    def drain(ci, k):
        for cp in copies(ci, k):
            cp.wait()

    accs = []
    for _ in range(8):
        accs += [jnp.full((L,), -1.0, jnp.float32),
                 jnp.ones((L,), jnp.float32),
                 jnp.zeros((L,), jnp.int32)]

    issue(0, 0)
    issue(1, 1)

    def do_chunk(ci, k, carry):
        drain(ci, k)
        tb, db, qb = tbufs[k], dbufs[k], qbufs[k]
        base = rs + ci * CROWS

        def tbody(kt, pc):
            pc = list(pc)
            for r in range(8):
                vrow = jnp.full((L,), base + kt * 8 + r, jnp.int32)
                for p in range(8):
                    t = tb[kt, r, pl.ds(p * L, L)]
                    d = db[kt, r, pl.ds(p * L, L)]
                    qv = qb[kt, r, pl.ds(p * L, L)]
                    diff = jnp.maximum(t - d, 0.0)
                    bd, bq, bi = pc[3 * p], pc[3 * p + 1], pc[3 * p + 2]
                    better = diff * bq > bd * qv
                    pc[3 * p] = jnp.where(better, diff, bd)
                    pc[3 * p + 1] = jnp.where(better, qv, bq)
                    pc[3 * p + 2] = jnp.where(better, vrow, bi)
            return tuple(pc)

        return list(lax.fori_loop(0, NT, tbody, tuple(carry)))

    def two_chunks(i, carry):
        carry = do_chunk(2 * i, 0, carry)

        @pl.when(2 * i + 2 < SC_NCH)
        def _():
            issue(2 * i + 2, 0)

        carry = do_chunk(2 * i + 1, 1, carry)

        @pl.when(2 * i + 3 < SC_NCH)
        def _():
            issue(2 * i + 3, 1)
        return tuple(carry)

    accs = list(lax.fori_loop(0, SC_NCH // 2, two_chunks, tuple(accs)))

    for p in range(8):
        bd, bq, bi = accs[3 * p], accs[3 * p + 1], accs[3 * p + 2]
        obuf_v[pl.ds(p * L, L)] = bd / bq
        obi_v[pl.ds(p * L, L)] = bi
    pltpu.sync_copy(obuf_v, oval.at[pl.ds(w * NROW, NROW)])
    pltpu.sync_copy(obi_v, oidx.at[pl.ds(w * NROW, NROW)])

    k4 = jnp.minimum(iota & 7, SPEC - 1)
    toks = w * SPEC + k4
    idxg_v[...] = plsc.load_gather(ids_v, [toks])
    pltpu.async_copy(t_hbm.at[idxg_v], gt_v, gsem).wait()
    pltpu.async_copy(d_hbm.at[idxg_v], gd_v, gsem).wait()
    gd = plsc.load_gather(gd_v, [iota, toks])
    gt = plsc.load_gather(gt_v, [iota, toks])
    pp = jnp.where(iota < 8, gd, gt)
    tmpf_v[...] = pp
    pltpu.sync_copy(tmpf_v, opp.at[pl.ds(w * L, L)])


@functools.lru_cache(maxsize=1)
def _build_sc():
    mesh = plsc.VectorSubcoreMesh(core_axis_name="c", subcore_axis_name="s")
    return pl.kernel(
        _sc_body,
        out_type=(
            jax.ShapeDtypeStruct((32 * NROW,), jnp.float32),
            jax.ShapeDtypeStruct((32 * NROW,), jnp.int32),
            jax.ShapeDtypeStruct((32 * L,), jnp.float32),
        ),
        mesh=mesh,
        compiler_params=pltpu.CompilerParams(needs_layout_passes=False),
        scratch_types=[
            pltpu.VMEM((NT, 8, 128), jnp.float32),
            pltpu.VMEM((NT, 8, 128), jnp.float32),
            pltpu.VMEM((NT, 8, 128), jnp.float32),
            pltpu.VMEM((NT, 8, 128), jnp.float32),
            pltpu.VMEM((NT, 8, 128), jnp.float32),
            pltpu.VMEM((NT, 8, 128), jnp.float32),
            pltpu.VMEM((NROW,), jnp.int32),
            pltpu.VMEM((L,), jnp.float32),
            pltpu.VMEM((NROW,), jnp.float32),
            pltpu.VMEM((NROW,), jnp.int32),
            pltpu.VMEM((L,), jnp.int32),
            pltpu.VMEM((L, 128), jnp.float32),
            pltpu.VMEM((L, 128), jnp.float32),
            pltpu.SemaphoreType.DMA,
            pltpu.SemaphoreType.DMA,
            pltpu.SemaphoreType.DMA,
        ],
    )


def _tc_scan_body(t_ref, d_ref, q_ref, m_ref, i_ref):
    g = pl.program_id(0)
    base = g * TC_B0
    val = jnp.maximum(t_ref[...] - d_ref[...], 0.0) / q_ref[...]
    cm = jnp.max(val, axis=0, keepdims=True)
    ri = lax.broadcasted_iota(jnp.int32, (TC_B0, NROW), 0)
    ca = jnp.min(jnp.where(val == cm, ri, INT_MAX), axis=0,
                 keepdims=True) + base

    first = g == 0
    m_prev = jnp.where(first, -1.0, m_ref[...])
    i_prev = jnp.where(first, 0, i_ref[...])
    better = cm > m_prev
    m_ref[...] = jnp.where(better, cm, m_prev)
    i_ref[...] = jnp.where(better, ca, i_prev)


def _tc_scan(t, d, q):
    blk = pl.BlockSpec((TC_B0, NROW), lambda i: (i, 0))
    vec = pl.BlockSpec((1, NROW), lambda i: (0, 0))
    return pl.pallas_call(
        _tc_scan_body,
        grid=(TC_NG,),
        in_specs=[blk, blk, blk],
        out_specs=[vec, vec],
        out_shape=[
            jax.ShapeDtypeStruct((1, NROW), jnp.float32),
            jax.ShapeDtypeStruct((1, NROW), jnp.int32),
        ],
        compiler_params=pltpu.CompilerParams(
            dimension_semantics=("arbitrary",)),
    )(t, d, q)


def _m1_body(tcm, tci, scv, sci, rec_ref):
    vm = jnp.max(scv[...], axis=0, keepdims=True)
    im = jnp.min(jnp.where(scv[...] == vm, sci[...], INT_MAX), axis=0,
                 keepdims=True)
    best_tc = tcm[...] >= vm
    rec_ref[...] = jnp.where(best_tc, tci[...], im)


def _merge1(tcm, tci, scv, sci):
    vec = pl.BlockSpec((1, NROW), lambda i: (0, 0))
    m32 = pl.BlockSpec((32, NROW), lambda i: (0, 0))
    return pl.pallas_call(
        _m1_body,
        grid=(1,),
        in_specs=[vec, vec, m32, m32],
        out_specs=vec,
        out_shape=jax.ShapeDtypeStruct((1, NROW), jnp.int32),
    )(tcm, tci, scv, sci)


def _m2_body(rec, dp, tp, ids, uni, bon, gre, out_ref):
    accm = (dp[...] > 0.0) & ((tp[...] / jnp.maximum(dp[...], 1e-30))
                              >= uni[...])
    token = jnp.where(accm, ids[...], rec[...])
    cols = []
    cum = jnp.ones((BATCH, 1), jnp.bool_)
    for k in range(SPEC):
        cols.append(jnp.where(cum, token[:, k:k + 1], PLACEHOLDER))
        cum = cum & accm[:, k:k + 1]
    cols.append(jnp.where(cum, bon[...], PLACEHOLDER))
    out = jnp.concatenate(cols, axis=1)
    out_ref[...] = jnp.where(gre[...] > 0, PLACEHOLDER, out)


def _merge2(*args):
    m4 = pl.BlockSpec((BATCH, SPEC), lambda i: (0, 0))
    m1 = pl.BlockSpec((BATCH, 1), lambda i: (0, 0))
    return pl.pallas_call(
        _m2_body,
        grid=(1,),
        in_specs=[m4, m4, m4, m4, m4, m1, m1],
        out_specs=pl.BlockSpec((BATCH, SPEC + 1), lambda i: (0, 0)),
        out_shape=jax.ShapeDtypeStruct((BATCH, SPEC + 1), jnp.int32),
    )(*args)


def kernel(draft_token_ids, draft_probs, target_probs, bonus_token_ids,
           uniform_probs, q, cu_num_draft_tokens, is_greedy):
    del cu_num_draft_tokens
    ids = draft_token_ids.astype(jnp.int32)
    bon = bonus_token_ids.reshape(-1).astype(jnp.int32)
    gre = is_greedy.astype(jnp.int32)

    tT = target_probs.T
    dT = draft_probs.T
    qT = q.T

    oval, oidx, opp = _build_sc()(tT, dT, qT, ids)
    scv = oval.reshape(32, NROW)
    sci = oidx.reshape(32, NROW)
    pp = opp.reshape(32, L)

    tcm, tci = _tc_scan(tT, dT, qT)
    rec = _merge1(tcm, tci, scv, sci)

    out = _merge2(rec.reshape(BATCH, SPEC), pp[:, :SPEC], pp[:, 8:8 + SPEC],
                  ids.reshape(BATCH, SPEC), uniform_probs.reshape(BATCH, SPEC),
                  bon.reshape(BATCH, 1), gre.reshape(BATCH, 1))
    return out.astype(draft_token_ids.dtype)


# speedup vs baseline: 3.7347x; 1.0840x over previous
import functools

import jax
import jax.numpy as jnp
from jax import lax
from jax.experimental import pallas as pl
from jax.experimental.pallas import tpu as pltpu
from jax.experimental.pallas import tpu_sc as plsc

PLACEHOLDER = -1
VOCAB = 100000
SPEC = 4
BATCH = 32
NROW = BATCH * SPEC
L = 16
INT_MAX = 2**31 - 1

V_TC = 59040
TC_B0 = 3936
TC_NG = V_TC // TC_B0
SC_ROWS = VOCAB - V_TC
W_ROWS = SC_ROWS // 32
CROWS = 128
SC_NCH = W_ROWS // CROWS


def _sc_body(t_hbm, d_hbm, q_hbm, ids_hbm,
             oval, oidx, opp,
             tb0, tb1, db0, db1, qb0, qb1,
             ids_v, tmpf_v, obuf_v, obi_v, idxg_v, gt_v, gd_v,
             sem0, sem1, gsem):
    c = lax.axis_index("c")
    s = lax.axis_index("s")
    w = c * 16 + s
    rs = V_TC + w * W_ROWS
    iota = lax.iota(jnp.int32, L)

    pltpu.sync_copy(ids_hbm, ids_v)

    tbufs = (tb0, tb1)
    dbufs = (db0, db1)
    qbufs = (qb0, qb1)
    sems = (sem0, sem1)

    def copies(ci, k):
        r0 = pl.multiple_of(rs + ci * CROWS, 8)
        return (
            pltpu.make_async_copy(
                t_hbm.at[pl.ds(r0, CROWS)], tbufs[k], sems[k]),
            pltpu.make_async_copy(
                d_hbm.at[pl.ds(r0, CROWS)], dbufs[k], sems[k]),
            pltpu.make_async_copy(
                q_hbm.at[pl.ds(r0, CROWS)], qbufs[k], sems[k]),
        )

    def issue(ci, k):
        for cp in copies(ci, k):
            cp.start()

    def drain(ci, k):
        for cp in copies(ci, k):
            cp.wait()

    accs = []
    for _ in range(8):
        accs += [jnp.full((L,), -1.0, jnp.float32),
                 jnp.ones((L,), jnp.float32),
                 jnp.zeros((L,), jnp.int32)]

    issue(0, 0)
    issue(1, 1)

    def do_chunk(ci, k, carry):
        drain(ci, k)
        tb, db, qb = tbufs[k], dbufs[k], qbufs[k]
        base = rs + ci * CROWS

        def tbody(kt, pc):
            pc = list(pc)
            for r in range(8):
                row = kt * 8 + r
                vrow = jnp.full((L,), base + row, jnp.int32)
                for p in range(8):
                    t = tb[row, pl.ds(p * L, L)]
                    d = db[row, pl.ds(p * L, L)]
                    qv = qb[row, pl.ds(p * L, L)]
                    diff = jnp.maximum(t - d, 0.0)
                    bd, bq, bi = pc[3 * p], pc[3 * p + 1], pc[3 * p + 2]
                    better = diff * bq > bd * qv
                    pc[3 * p] = jnp.where(better, diff, bd)
                    pc[3 * p + 1] = jnp.where(better, qv, bq)
                    pc[3 * p + 2] = jnp.where(better, vrow, bi)
            return tuple(pc)

        return list(lax.fori_loop(0, CROWS // 8, tbody, tuple(carry)))

    def two_chunks(i, carry):
        carry = do_chunk(2 * i, 0, carry)

        @pl.when(2 * i + 2 < SC_NCH)
        def _():
            issue(2 * i + 2, 0)

        carry = do_chunk(2 * i + 1, 1, carry)

        @pl.when(2 * i + 3 < SC_NCH)
        def _():
            issue(2 * i + 3, 1)
        return tuple(carry)

    accs = list(lax.fori_loop(0, SC_NCH // 2, two_chunks, tuple(accs)))

    for p in range(8):
        bd, bq, bi = accs[3 * p], accs[3 * p + 1], accs[3 * p + 2]
        obuf_v[pl.ds(p * L, L)] = bd / bq
        obi_v[pl.ds(p * L, L)] = bi
    pltpu.sync_copy(obuf_v, oval.at[pl.ds(w * NROW, NROW)])
    pltpu.sync_copy(obi_v, oidx.at[pl.ds(w * NROW, NROW)])

    k4 = jnp.minimum(iota & 7, SPEC - 1)
    toks = w * SPEC + k4
    idxg_v[...] = plsc.load_gather(ids_v, [toks])
    pltpu.async_copy(t_hbm.at[idxg_v], gt_v, gsem).wait()
    pltpu.async_copy(d_hbm.at[idxg_v], gd_v, gsem).wait()
    gd = plsc.load_gather(gd_v, [iota, toks])
    gt = plsc.load_gather(gt_v, [iota, toks])
    pp = jnp.where(iota < 8, gd, gt)
    tmpf_v[...] = pp
    pltpu.sync_copy(tmpf_v, opp.at[pl.ds(w * L, L)])


@functools.lru_cache(maxsize=1)
def _build_sc():
    mesh = plsc.VectorSubcoreMesh(core_axis_name="c", subcore_axis_name="s")
    return pl.kernel(
        _sc_body,
        out_type=(
            jax.ShapeDtypeStruct((32 * NROW,), jnp.float32),
            jax.ShapeDtypeStruct((32 * NROW,), jnp.int32),
            jax.ShapeDtypeStruct((32 * L,), jnp.float32),
        ),
        mesh=mesh,
        compiler_params=pltpu.CompilerParams(needs_layout_passes=False),
        scratch_types=[
            pltpu.VMEM((CROWS, 128), jnp.float32),
            pltpu.VMEM((CROWS, 128), jnp.float32),
            pltpu.VMEM((CROWS, 128), jnp.float32),
            pltpu.VMEM((CROWS, 128), jnp.float32),
            pltpu.VMEM((CROWS, 128), jnp.float32),
            pltpu.VMEM((CROWS, 128), jnp.float32),
            pltpu.VMEM((NROW,), jnp.int32),
            pltpu.VMEM((L,), jnp.float32),
            pltpu.VMEM((NROW,), jnp.float32),
            pltpu.VMEM((NROW,), jnp.int32),
            pltpu.VMEM((L,), jnp.int32),
            pltpu.VMEM((L, 128), jnp.float32),
            pltpu.VMEM((L, 128), jnp.float32),
            pltpu.SemaphoreType.DMA,
            pltpu.SemaphoreType.DMA,
            pltpu.SemaphoreType.DMA,
        ],
    )


def _tc_scan_body(t_ref, d_ref, q_ref, m_ref, i_ref):
    g = pl.program_id(0)
    base = g * TC_B0
    val = jnp.maximum(t_ref[...] - d_ref[...], 0.0) / q_ref[...]
    cm = jnp.max(val, axis=0, keepdims=True)
    ri = lax.broadcasted_iota(jnp.int32, (TC_B0, NROW), 0)
    ca = jnp.min(jnp.where(val == cm, ri, INT_MAX), axis=0,
                 keepdims=True) + base

    first = g == 0
    m_prev = jnp.where(first, -1.0, m_ref[...])
    i_prev = jnp.where(first, 0, i_ref[...])
    better = cm > m_prev
    m_ref[...] = jnp.where(better, cm, m_prev)
    i_ref[...] = jnp.where(better, ca, i_prev)


def _tc_scan(t, d, q):
    blk = pl.BlockSpec((TC_B0, NROW), lambda i: (i, 0))
    vec = pl.BlockSpec((1, NROW), lambda i: (0, 0))
    return pl.pallas_call(
        _tc_scan_body,
        grid=(TC_NG,),
        in_specs=[blk, blk, blk],
        out_specs=[vec, vec],
        out_shape=[
            jax.ShapeDtypeStruct((1, NROW), jnp.float32),
            jax.ShapeDtypeStruct((1, NROW), jnp.int32),
        ],
        compiler_params=pltpu.CompilerParams(
            dimension_semantics=("arbitrary",)),
    )(t, d, q)


def _m1_body(tcm, tci, scv, sci, rec_ref):
    vm = jnp.max(scv[...], axis=0, keepdims=True)
    im = jnp.min(jnp.where(scv[...] == vm, sci[...], INT_MAX), axis=0,
                 keepdims=True)
    best_tc = tcm[...] >= vm
    rec_ref[...] = jnp.where(best_tc, tci[...], im)


def _merge1(tcm, tci, scv, sci):
    vec = pl.BlockSpec((1, NROW), lambda i: (0, 0))
    m32 = pl.BlockSpec((32, NROW), lambda i: (0, 0))
    return pl.pallas_call(
        _m1_body,
        grid=(1,),
        in_specs=[vec, vec, m32, m32],
        out_specs=vec,
        out_shape=jax.ShapeDtypeStruct((1, NROW), jnp.int32),
    )(tcm, tci, scv, sci)


def _m2_body(rec, dp, tp, ids, uni, bon, gre, out_ref):
    accm = (dp[...] > 0.0) & ((tp[...] / jnp.maximum(dp[...], 1e-30))
                              >= uni[...])
    token = jnp.where(accm, ids[...], rec[...])
    cols = []
    cum = jnp.ones((BATCH, 1), jnp.bool_)
    for k in range(SPEC):
        cols.append(jnp.where(cum, token[:, k:k + 1], PLACEHOLDER))
        cum = cum & accm[:, k:k + 1]
    cols.append(jnp.where(cum, bon[...], PLACEHOLDER))
    out = jnp.concatenate(cols, axis=1)
    out_ref[...] = jnp.where(gre[...] > 0, PLACEHOLDER, out)


def _merge2(*args):
    m4 = pl.BlockSpec((BATCH, SPEC), lambda i: (0, 0))
    m1 = pl.BlockSpec((BATCH, 1), lambda i: (0, 0))
    return pl.pallas_call(
        _m2_body,
        grid=(1,),
        in_specs=[m4, m4, m4, m4, m4, m1, m1],
        out_specs=pl.BlockSpec((BATCH, SPEC + 1), lambda i: (0, 0)),
        out_shape=jax.ShapeDtypeStruct((BATCH, SPEC + 1), jnp.int32),
    )(*args)


def kernel(draft_token_ids, draft_probs, target_probs, bonus_token_ids,
           uniform_probs, q, cu_num_draft_tokens, is_greedy):
    del cu_num_draft_tokens
    ids = draft_token_ids.astype(jnp.int32)
    bon = bonus_token_ids.reshape(-1).astype(jnp.int32)
    gre = is_greedy.astype(jnp.int32)

    tT = target_probs.T
    dT = draft_probs.T
    qT = q.T

    oval, oidx, opp = _build_sc()(tT, dT, qT, ids)
    scv = oval.reshape(32, NROW)
    sci = oidx.reshape(32, NROW)
    pp = opp.reshape(32, L)

    tcm, tci = _tc_scan(tT, dT, qT)
    rec = _merge1(tcm, tci, scv, sci)

    out = _merge2(rec.reshape(BATCH, SPEC), pp[:, :SPEC], pp[:, 8:8 + SPEC],
                  ids.reshape(BATCH, SPEC), uniform_probs.reshape(BATCH, SPEC),
                  bon.reshape(BATCH, 1), gre.reshape(BATCH, 1))
    return out.astype(draft_token_ids.dtype)


# speedup vs baseline: 4.8273x; 1.2926x over previous
import functools

import jax
import jax.numpy as jnp
from jax import lax
from jax.experimental import pallas as pl
from jax.experimental.pallas import tpu as pltpu
from jax.experimental.pallas import tpu_sc as plsc

PLACEHOLDER = -1
VOCAB = 100000
SPEC = 4
BATCH = 32
NROW = BATCH * SPEC
L = 16
INT_MAX = 2**31 - 1

V_TC = 76800
TC_B0 = 3840
TC_NG = V_TC // TC_B0
SC_ROWS = 24576
S0 = VOCAB - SC_ROWS
W_ROWS = SC_ROWS // 32
CROWS = 128
SC_NCH = W_ROWS // CROWS


def _sc_body(t_hbm, d_hbm, q_hbm, ids_hbm,
             oval, oidx, opp,
             tb0, tb1, db0, db1, qb0, qb1,
             ids_v, tmpf_v, obuf_v, obi_v, idxg_v, gt_v, gd_v,
             sem0, sem1, gsem):
    c = lax.axis_index("c")
    s = lax.axis_index("s")
    w = c * 16 + s
    rs = S0 + w * W_ROWS
    iota = lax.iota(jnp.int32, L)

    pltpu.sync_copy(ids_hbm, ids_v)

    tbufs = (tb0, tb1)
    dbufs = (db0, db1)
    qbufs = (qb0, qb1)
    sems = (sem0, sem1)

    def copies(ci, k):
        r0 = pl.multiple_of(rs + ci * CROWS, 8)
        return (
            pltpu.make_async_copy(
                t_hbm.at[pl.ds(r0, CROWS)], tbufs[k], sems[k]),
            pltpu.make_async_copy(
                d_hbm.at[pl.ds(r0, CROWS)], dbufs[k], sems[k]),
            pltpu.make_async_copy(
                q_hbm.at[pl.ds(r0, CROWS)], qbufs[k], sems[k]),
        )

    def issue(ci, k):
        for cp in copies(ci, k):
            cp.start()

    def drain(ci, k):
        for cp in copies(ci, k):
            cp.wait()

    accs = []
    for _ in range(8):
        accs += [jnp.full((L,), -1.0, jnp.float32),
                 jnp.ones((L,), jnp.float32),
                 jnp.zeros((L,), jnp.int32)]

    issue(0, 0)
    issue(1, 1)

    def do_chunk(ci, k, carry):
        drain(ci, k)
        tb, db, qb = tbufs[k], dbufs[k], qbufs[k]
        base = rs + ci * CROWS

        def tbody(kt, pc):
            pc = list(pc)
            for r in range(8):
                row = kt * 8 + r
                vrow = jnp.full((L,), base + row, jnp.int32)
                for p in range(8):
                    t = tb[row, pl.ds(p * L, L)]
                    d = db[row, pl.ds(p * L, L)]
                    qv = qb[row, pl.ds(p * L, L)]
                    diff = jnp.maximum(t - d, 0.0)
                    bd, bq, bi = pc[3 * p], pc[3 * p + 1], pc[3 * p + 2]
                    better = diff * bq > bd * qv
                    pc[3 * p] = jnp.where(better, diff, bd)
                    pc[3 * p + 1] = jnp.where(better, qv, bq)
                    pc[3 * p + 2] = jnp.where(better, vrow, bi)
            return tuple(pc)

        return list(lax.fori_loop(0, CROWS // 8, tbody, tuple(carry)))

    def two_chunks(i, carry):
        carry = do_chunk(2 * i, 0, carry)

        @pl.when(2 * i + 2 < SC_NCH)
        def _():
            issue(2 * i + 2, 0)

        carry = do_chunk(2 * i + 1, 1, carry)

        @pl.when(2 * i + 3 < SC_NCH)
        def _():
            issue(2 * i + 3, 1)
        return tuple(carry)

    accs = list(lax.fori_loop(0, SC_NCH // 2, two_chunks, tuple(accs)))

    for p in range(8):
        bd, bq, bi = accs[3 * p], accs[3 * p + 1], accs[3 * p + 2]
        obuf_v[pl.ds(p * L, L)] = bd / bq
        obi_v[pl.ds(p * L, L)] = bi
    pltpu.sync_copy(obuf_v, oval.at[pl.ds(w * NROW, NROW)])
    pltpu.sync_copy(obi_v, oidx.at[pl.ds(w * NROW, NROW)])

    k4 = jnp.minimum(iota & 7, SPEC - 1)
    toks = w * SPEC + k4
    idxg_v[...] = plsc.load_gather(ids_v, [toks])
    pltpu.async_copy(t_hbm.at[idxg_v], gt_v, gsem).wait()
    pltpu.async_copy(d_hbm.at[idxg_v], gd_v, gsem).wait()
    gd = plsc.load_gather(gd_v, [iota, toks])
    gt = plsc.load_gather(gt_v, [iota, toks])
    pp = jnp.where(iota < 8, gd, gt)
    tmpf_v[...] = pp
    pltpu.sync_copy(tmpf_v, opp.at[pl.ds(w * L, L)])


@functools.lru_cache(maxsize=1)
def _build_sc():
    mesh = plsc.VectorSubcoreMesh(core_axis_name="c", subcore_axis_name="s")
    return pl.kernel(
        _sc_body,
        out_type=(
            jax.ShapeDtypeStruct((32 * NROW,), jnp.float32),
            jax.ShapeDtypeStruct((32 * NROW,), jnp.int32),
            jax.ShapeDtypeStruct((32 * L,), jnp.float32),
        ),
        mesh=mesh,
        compiler_params=pltpu.CompilerParams(needs_layout_passes=False),
        scratch_types=[
            pltpu.VMEM((CROWS, 128), jnp.float32),
            pltpu.VMEM((CROWS, 128), jnp.float32),
            pltpu.VMEM((CROWS, 128), jnp.float32),
            pltpu.VMEM((CROWS, 128), jnp.float32),
            pltpu.VMEM((CROWS, 128), jnp.float32),
            pltpu.VMEM((CROWS, 128), jnp.float32),
            pltpu.VMEM((NROW,), jnp.int32),
            pltpu.VMEM((L,), jnp.float32),
            pltpu.VMEM((NROW,), jnp.float32),
            pltpu.VMEM((NROW,), jnp.int32),
            pltpu.VMEM((L,), jnp.int32),
            pltpu.VMEM((L, 128), jnp.float32),
            pltpu.VMEM((L, 128), jnp.float32),
            pltpu.SemaphoreType.DMA,
            pltpu.SemaphoreType.DMA,
            pltpu.SemaphoreType.DMA,
        ],
    )


def _tc_scan_body(t_ref, d_ref, q_ref, m_ref, i_ref):
    g = pl.program_id(0)
    base = g * TC_B0
    val = jnp.maximum(t_ref[...] - d_ref[...], 0.0) / q_ref[...]
    cm = jnp.max(val, axis=0, keepdims=True)
    ri = lax.broadcasted_iota(jnp.int32, (TC_B0, NROW), 0)
    ca = jnp.min(jnp.where(val == cm, ri, INT_MAX), axis=0,
                 keepdims=True) + base

    first = g == 0
    m_prev = jnp.where(first, -1.0, m_ref[...])
    i_prev = jnp.where(first, 0, i_ref[...])
    better = cm > m_prev
    m_ref[...] = jnp.where(better, cm, m_prev)
    i_ref[...] = jnp.where(better, ca, i_prev)


def _tc_scan(t, d, q):
    blk = pl.BlockSpec((TC_B0, NROW), lambda i: (i, 0))
    vec = pl.BlockSpec((1, NROW), lambda i: (0, 0))
    return pl.pallas_call(
        _tc_scan_body,
        grid=(TC_NG,),
        in_specs=[blk, blk, blk],
        out_specs=[vec, vec],
        out_shape=[
            jax.ShapeDtypeStruct((1, NROW), jnp.float32),
            jax.ShapeDtypeStruct((1, NROW), jnp.int32),
        ],
        compiler_params=pltpu.CompilerParams(
            dimension_semantics=("arbitrary",)),
    )(t, d, q)


def _m1_body(tcm, tci, scv, sci, rec_ref):
    vm = jnp.max(scv[...], axis=0, keepdims=True)
    im = jnp.min(jnp.where(scv[...] == vm, sci[...], INT_MAX), axis=0,
                 keepdims=True)
    best_tc = tcm[...] >= vm
    rec_ref[...] = jnp.where(best_tc, tci[...], im)


def _merge1(tcm, tci, scv, sci):
    vec = pl.BlockSpec((1, NROW), lambda i: (0, 0))
    m32 = pl.BlockSpec((32, NROW), lambda i: (0, 0))
    return pl.pallas_call(
        _m1_body,
        grid=(1,),
        in_specs=[vec, vec, m32, m32],
        out_specs=vec,
        out_shape=jax.ShapeDtypeStruct((1, NROW), jnp.int32),
    )(tcm, tci, scv, sci)


def _m2_body(rec, dp, tp, ids, uni, bon, gre, out_ref):
    accm = (dp[...] > 0.0) & ((tp[...] / jnp.maximum(dp[...], 1e-30))
                              >= uni[...])
    token = jnp.where(accm, ids[...], rec[...])
    cols = []
    cum = jnp.ones((BATCH, 1), jnp.bool_)
    for k in range(SPEC):
        cols.append(jnp.where(cum, token[:, k:k + 1], PLACEHOLDER))
        cum = cum & accm[:, k:k + 1]
    cols.append(jnp.where(cum, bon[...], PLACEHOLDER))
    out = jnp.concatenate(cols, axis=1)
    out_ref[...] = jnp.where(gre[...] > 0, PLACEHOLDER, out)


def _merge2(*args):
    m4 = pl.BlockSpec((BATCH, SPEC), lambda i: (0, 0))
    m1 = pl.BlockSpec((BATCH, 1), lambda i: (0, 0))
    return pl.pallas_call(
        _m2_body,
        grid=(1,),
        in_specs=[m4, m4, m4, m4, m4, m1, m1],
        out_specs=pl.BlockSpec((BATCH, SPEC + 1), lambda i: (0, 0)),
        out_shape=jax.ShapeDtypeStruct((BATCH, SPEC + 1), jnp.int32),
    )(*args)


def kernel(draft_token_ids, draft_probs, target_probs, bonus_token_ids,
           uniform_probs, q, cu_num_draft_tokens, is_greedy):
    del cu_num_draft_tokens
    ids = draft_token_ids.astype(jnp.int32)
    bon = bonus_token_ids.reshape(-1).astype(jnp.int32)
    gre = is_greedy.astype(jnp.int32)

    tT = target_probs.T
    dT = draft_probs.T
    qT = q.T

    oval, oidx, opp = _build_sc()(tT, dT, qT, ids)
    scv = oval.reshape(32, NROW)
    sci = oidx.reshape(32, NROW)
    pp = opp.reshape(32, L)

    tcm, tci = _tc_scan(tT, dT, qT)
    rec = _merge1(tcm, tci, scv, sci)

    out = _merge2(rec.reshape(BATCH, SPEC), pp[:, :SPEC], pp[:, 8:8 + SPEC],
                  ids.reshape(BATCH, SPEC), uniform_probs.reshape(BATCH, SPEC),
                  bon.reshape(BATCH, 1), gre.reshape(BATCH, 1))
    return out.astype(draft_token_ids.dtype)
